# baseline (speedup 1.0000x reference)
import functools

import jax
import jax.numpy as jnp
from jax.experimental import pallas as pl
from jax.experimental.pallas import tpu as pltpu

B, N, D, OUT = 4, 100000, 128, 10
TILE_N = 5000
NT = N // TILE_N


def _fused_kernel(h_ref, w_in_ref, w_h1_ref, w_h2_ref, w_out_ref,
                  c1w_ref, c1b_ref, c2w_ref, c2b_ref, c3w_ref, c3b_ref,
                  out_ref, pooled_ref):
    b = pl.program_id(0)
    nt = pl.program_id(1)

    @pl.when((b == 0) & (nt == 0))
    def _init():
        pooled_ref[:, :] = jnp.zeros((8, D), jnp.float32)

    x = h_ref[0].astype(jnp.bfloat16)
    h = jnp.maximum(jnp.dot(x, w_in_ref[:, :], preferred_element_type=jnp.float32), 0.0)
    h = jnp.maximum(jnp.dot(h.astype(jnp.bfloat16), w_h1_ref[:, :], preferred_element_type=jnp.float32), 0.0)
    h = jnp.maximum(jnp.dot(h.astype(jnp.bfloat16), w_h2_ref[:, :], preferred_element_type=jnp.float32), 0.0)
    h = jnp.maximum(jnp.dot(h.astype(jnp.bfloat16), w_out_ref[:, :], preferred_element_type=jnp.float32), 0.0)
    partial = jnp.sum(h, axis=0, keepdims=True)

    rows = jax.lax.broadcasted_iota(jnp.int32, (8, D), 0)
    pooled_ref[:, :] = jnp.where(rows == b, pooled_ref[:, :] + partial,
                                 pooled_ref[:, :])

    @pl.when((b == B - 1) & (nt == NT - 1))
    def _classify():
        acc = pooled_ref[0:B, :]
        y = jnp.maximum(jnp.dot(acc, c1w_ref[:, :],
                                preferred_element_type=jnp.float32)
                        + c1b_ref[:, :], 0.0)
        y = jnp.maximum(jnp.dot(y, c2w_ref[:, :],
                                preferred_element_type=jnp.float32)
                        + c2b_ref[:, :], 0.0)
        y = (jnp.dot(y, c3w_ref[:, :], preferred_element_type=jnp.float32)
             + c3b_ref[:, :])
        out_ref[:, :] = y


@functools.partial(jax.jit, static_argnames=())
def kernel(h_0, W_in, W_h1, W_h2, W_out, C1_w, C1_b, C2_w, C2_b, C3_w, C3_b):
    const = lambda shape: pl.BlockSpec(shape, lambda b, n: (0,) * len(shape))
    return pl.pallas_call(
        _fused_kernel,
        grid=(B, NT),
        in_specs=[
            pl.BlockSpec((1, TILE_N, D), lambda b, n: (b, n, 0)),
            const((D, D)), const((D, D)), const((D, D)), const((D, D)),
            const((D, D)), const((1, D)),
            const((D, D)), const((1, D)),
            const((D, OUT)), const((1, OUT)),
        ],
        out_specs=const((B, OUT)),
        out_shape=jax.ShapeDtypeStruct((B, OUT), jnp.float32),
        scratch_shapes=[pltpu.VMEM((8, D), jnp.float32)],
        compiler_params=pltpu.CompilerParams(
            dimension_semantics=("arbitrary", "arbitrary")),
    )(h_0, W_in.astype(jnp.bfloat16), W_h1.astype(jnp.bfloat16),
      W_h2.astype(jnp.bfloat16), W_out.astype(jnp.bfloat16),
      C1_w, C1_b.reshape(1, D), C2_w, C2_b.reshape(1, D),
      C3_w, C3_b.reshape(1, OUT))


# speedup vs baseline: 1.0540x; 1.0540x over previous
import jax
import jax.numpy as jnp
from jax.experimental import pallas as pl
from jax.experimental.pallas import tpu as pltpu

B, N, D, OUT = 4, 100000, 128, 10
TILE_N = 10000
NT = N // TILE_N


def _gcn_pool_kernel(h_ref, w_in_ref, w_h1_ref, w_h2_ref, w_out_ref,
                     pooled_ref):
    b = pl.program_id(0)
    nt = pl.program_id(1)

    @pl.when((b == 0) & (nt == 0))
    def _init():
        pooled_ref[:, :] = jnp.zeros((8, D), jnp.float32)

    x = h_ref[0].astype(jnp.bfloat16)
    h = jnp.maximum(jnp.dot(x, w_in_ref[:, :],
                            preferred_element_type=jnp.float32), 0.0)
    h = jnp.maximum(jnp.dot(h.astype(jnp.bfloat16), w_h1_ref[:, :],
                            preferred_element_type=jnp.float32), 0.0)
    h = jnp.maximum(jnp.dot(h.astype(jnp.bfloat16), w_h2_ref[:, :],
                            preferred_element_type=jnp.float32), 0.0)
    h = jnp.maximum(jnp.dot(h.astype(jnp.bfloat16), w_out_ref[:, :],
                            preferred_element_type=jnp.float32), 0.0)
    partial = jnp.sum(h, axis=0, keepdims=True)

    rows = jax.lax.broadcasted_iota(jnp.int32, (8, D), 0)
    pooled_ref[:, :] = jnp.where(rows == b, pooled_ref[:, :] + partial,
                                 pooled_ref[:, :])


def _mlp_kernel(pooled_ref, c1w_ref, c1b_ref, c2w_ref, c2b_ref,
                c3w_ref, c3b_ref, out_ref):
    acc = pooled_ref[0:B, :]
    y = jnp.maximum(jnp.dot(acc, c1w_ref[:, :],
                            preferred_element_type=jnp.float32)
                    + c1b_ref[:, :], 0.0)
    y = jnp.maximum(jnp.dot(y, c2w_ref[:, :],
                            preferred_element_type=jnp.float32)
                    + c2b_ref[:, :], 0.0)
    out_ref[:, :] = (jnp.dot(y, c3w_ref[:, :],
                             preferred_element_type=jnp.float32)
                     + c3b_ref[:, :])


def kernel(h_0, W_in, W_h1, W_h2, W_out, C1_w, C1_b, C2_w, C2_b, C3_w, C3_b):
    const = lambda shape: pl.BlockSpec(shape, lambda b, n: (0,) * len(shape))
    pooled = pl.pallas_call(
        _gcn_pool_kernel,
        grid=(B, NT),
        in_specs=[
            pl.BlockSpec((1, TILE_N, D), lambda b, n: (b, n, 0)),
            const((D, D)), const((D, D)), const((D, D)), const((D, D)),
        ],
        out_specs=const((8, D)),
        out_shape=jax.ShapeDtypeStruct((8, D), jnp.float32),
        compiler_params=pltpu.CompilerParams(
            dimension_semantics=("arbitrary", "arbitrary")),
    )(h_0, W_in.astype(jnp.bfloat16), W_h1.astype(jnp.bfloat16),
      W_h2.astype(jnp.bfloat16), W_out.astype(jnp.bfloat16))

    return pl.pallas_call(
        _mlp_kernel,
        in_specs=[
            pl.BlockSpec((8, D), lambda: (0, 0)),
            pl.BlockSpec((D, D), lambda: (0, 0)),
            pl.BlockSpec((1, D), lambda: (0, 0)),
            pl.BlockSpec((D, D), lambda: (0, 0)),
            pl.BlockSpec((1, D), lambda: (0, 0)),
            pl.BlockSpec((D, OUT), lambda: (0, 0)),
            pl.BlockSpec((1, OUT), lambda: (0, 0)),
        ],
        out_specs=pl.BlockSpec((B, OUT), lambda: (0, 0)),
        out_shape=jax.ShapeDtypeStruct((B, OUT), jnp.float32),
    )(pooled, C1_w, C1_b.reshape(1, D), C2_w, C2_b.reshape(1, D),
      C3_w, C3_b.reshape(1, OUT))


# speedup vs baseline: 1.3192x; 1.2516x over previous
import jax
import jax.numpy as jnp
from jax.experimental import pallas as pl
from jax.experimental.pallas import tpu as pltpu

B, N, D, OUT = 4, 100000, 128, 10
TILE_N = 10000
NT = N // TILE_N
BNT = B * NT


def _gcn_pool_kernel(h_ref, wab_ref, wcd_ref, pooled_ref, c1_ref, c3_ref):
    s = pl.program_id(0)

    @pl.when(s == 0)
    def _init():
        pooled_ref[:, :] = jnp.zeros((8, D), jnp.float32)
        c1_ref[:, :] = jnp.zeros((TILE_N, D), jnp.bfloat16)
        c3_ref[:, :] = jnp.zeros((TILE_N, D), jnp.bfloat16)

    x = h_ref[0].astype(jnp.bfloat16)
    in_a = jnp.concatenate([x, c1_ref[:, :]], axis=1)
    o_a = jnp.maximum(jnp.dot(in_a, wab_ref[:, :],
                              preferred_element_type=jnp.float32), 0.0)
    h1_s = o_a[:, :D]
    h2_p = o_a[:, D:]

    in_b = jnp.concatenate([h2_p.astype(jnp.bfloat16), c3_ref[:, :]], axis=1)
    o_b = jnp.maximum(jnp.dot(in_b, wcd_ref[:, :],
                              preferred_element_type=jnp.float32), 0.0)
    h3_p = o_b[:, :D]
    h4_pp = o_b[:, D:]

    c1_ref[:, :] = h1_s.astype(jnp.bfloat16)
    c3_ref[:, :] = h3_p.astype(jnp.bfloat16)

    @pl.when(s >= 2)
    def _pool():
        partial = jnp.sum(h4_pp, axis=0, keepdims=True)
        b4 = (s - 2) // NT
        rows = jax.lax.broadcasted_iota(jnp.int32, (8, D), 0)
        pooled_ref[:, :] = jnp.where(rows == b4, pooled_ref[:, :] + partial,
                                     pooled_ref[:, :])


def _mlp_kernel(pooled_ref, c1w_ref, c1b_ref, c2w_ref, c2b_ref,
                c3w_ref, c3b_ref, out_ref):
    acc = pooled_ref[0:B, :]
    y = jnp.maximum(jnp.dot(acc, c1w_ref[:, :],
                            preferred_element_type=jnp.float32)
                    + c1b_ref[:, :], 0.0)
    y = jnp.maximum(jnp.dot(y, c2w_ref[:, :],
                            preferred_element_type=jnp.float32)
                    + c2b_ref[:, :], 0.0)
    out_ref[:, :] = (jnp.dot(y, c3w_ref[:, :],
                             preferred_element_type=jnp.float32)
                     + c3b_ref[:, :])


def _blkdiag(w_top, w_bot):
    z = jnp.zeros((D, D), jnp.bfloat16)
    return jnp.concatenate(
        [jnp.concatenate([w_top.astype(jnp.bfloat16), z], axis=1),
         jnp.concatenate([z, w_bot.astype(jnp.bfloat16)], axis=1)], axis=0)


def kernel(h_0, W_in, W_h1, W_h2, W_out, C1_w, C1_b, C2_w, C2_b, C3_w, C3_b):
    w_ab = _blkdiag(W_in, W_h1)
    w_cd = _blkdiag(W_h2, W_out)

    def x_map(s):
        t = jnp.minimum(s, BNT - 1)
        return (t // NT, t % NT, 0)

    pooled = pl.pallas_call(
        _gcn_pool_kernel,
        grid=(BNT + 2,),
        in_specs=[
            pl.BlockSpec((1, TILE_N, D), x_map),
            pl.BlockSpec((2 * D, 2 * D), lambda s: (0, 0)),
            pl.BlockSpec((2 * D, 2 * D), lambda s: (0, 0)),
        ],
        out_specs=pl.BlockSpec((8, D), lambda s: (0, 0)),
        out_shape=jax.ShapeDtypeStruct((8, D), jnp.float32),
        scratch_shapes=[pltpu.VMEM((TILE_N, D), jnp.bfloat16),
                        pltpu.VMEM((TILE_N, D), jnp.bfloat16)],
        compiler_params=pltpu.CompilerParams(
            dimension_semantics=("arbitrary",)),
    )(h_0, w_ab, w_cd)

    return pl.pallas_call(
        _mlp_kernel,
        in_specs=[
            pl.BlockSpec((8, D), lambda: (0, 0)),
            pl.BlockSpec((D, D), lambda: (0, 0)),
            pl.BlockSpec((1, D), lambda: (0, 0)),
            pl.BlockSpec((D, D), lambda: (0, 0)),
            pl.BlockSpec((1, D), lambda: (0, 0)),
            pl.BlockSpec((D, OUT), lambda: (0, 0)),
            pl.BlockSpec((1, OUT), lambda: (0, 0)),
        ],
        out_specs=pl.BlockSpec((B, OUT), lambda: (0, 0)),
        out_shape=jax.ShapeDtypeStruct((B, OUT), jnp.float32),
    )(pooled, C1_w, C1_b.reshape(1, D), C2_w, C2_b.reshape(1, D),
      C3_w, C3_b.reshape(1, OUT))


# speedup vs baseline: 1.8593x; 1.4095x over previous
import jax
import jax.numpy as jnp
from jax.experimental import pallas as pl
from jax.experimental.pallas import tpu as pltpu

B, N, D, OUT = 4, 100000, 128, 10
TILE_N = 10000
NT = N // TILE_N
BNT = B * NT
NCH = 5
CH = TILE_N // NCH


def _gcn_pool_kernel(h_ref, wab_ref, wcd_ref, pooled_ref, c1_ref, c3_ref):
    s = pl.program_id(0)

    @pl.when(s == 0)
    def _init():
        pooled_ref[:, :] = jnp.zeros((8, D), jnp.float32)
        c1_ref[:, :] = jnp.zeros((TILE_N, D), jnp.bfloat16)
        c3_ref[:, :] = jnp.zeros((TILE_N, D), jnp.bfloat16)

    def stage(in_left, in_right, w_ref):
        inp = jnp.concatenate([in_left, in_right], axis=1)
        return jnp.dot(inp, w_ref[:, :], preferred_element_type=jnp.float32)

    partial = jnp.zeros((1, D), jnp.float32)
    h2s = []
    for i in range(NCH):
        r = slice(i * CH, (i + 1) * CH)
        x_i = h_ref[0, r, :].astype(jnp.bfloat16)
        o_a = jnp.maximum(stage(x_i, c1_ref[r, :], wab_ref)
                          .astype(jnp.bfloat16), 0.0)
        c1_ref[r, :] = o_a[:, :D]
        h2s.append(o_a[:, D:])
    for i in range(NCH):
        r = slice(i * CH, (i + 1) * CH)
        o_b = stage(h2s[i], c3_ref[r, :], wcd_ref)
        c3_ref[r, :] = jnp.maximum(o_b[:, :D].astype(jnp.bfloat16), 0.0)
        h4_i = jnp.maximum(o_b[:, D:], 0.0)
        partial = partial + jnp.sum(h4_i, axis=0, keepdims=True)

    @pl.when(s >= 2)
    def _pool():
        b4 = (s - 2) // NT
        rows = jax.lax.broadcasted_iota(jnp.int32, (8, D), 0)
        pooled_ref[:, :] = jnp.where(rows == b4, pooled_ref[:, :] + partial,
                                     pooled_ref[:, :])


def _mlp_kernel(pooled_ref, c1w_ref, c1b_ref, c2w_ref, c2b_ref,
                c3w_ref, c3b_ref, out_ref):
    acc = pooled_ref[0:B, :]
    y = jnp.maximum(jnp.dot(acc, c1w_ref[:, :],
                            preferred_element_type=jnp.float32)
                    + c1b_ref[:, :], 0.0)
    y = jnp.maximum(jnp.dot(y, c2w_ref[:, :],
                            preferred_element_type=jnp.float32)
                    + c2b_ref[:, :], 0.0)
    out_ref[:, :] = (jnp.dot(y, c3w_ref[:, :],
                             preferred_element_type=jnp.float32)
                     + c3b_ref[:, :])


def _blkdiag(w_top, w_bot):
    z = jnp.zeros((D, D), jnp.bfloat16)
    return jnp.concatenate(
        [jnp.concatenate([w_top.astype(jnp.bfloat16), z], axis=1),
         jnp.concatenate([z, w_bot.astype(jnp.bfloat16)], axis=1)], axis=0)


def kernel(h_0, W_in, W_h1, W_h2, W_out, C1_w, C1_b, C2_w, C2_b, C3_w, C3_b):
    w_ab = _blkdiag(W_in, W_h1)
    w_cd = _blkdiag(W_h2, W_out)

    def x_map(s):
        t = jnp.minimum(s, BNT - 1)
        return (t // NT, t % NT, 0)

    pooled = pl.pallas_call(
        _gcn_pool_kernel,
        grid=(BNT + 2,),
        in_specs=[
            pl.BlockSpec((1, TILE_N, D), x_map),
            pl.BlockSpec((2 * D, 2 * D), lambda s: (0, 0)),
            pl.BlockSpec((2 * D, 2 * D), lambda s: (0, 0)),
        ],
        out_specs=pl.BlockSpec((8, D), lambda s: (0, 0)),
        out_shape=jax.ShapeDtypeStruct((8, D), jnp.float32),
        scratch_shapes=[pltpu.VMEM((TILE_N, D), jnp.bfloat16),
                        pltpu.VMEM((TILE_N, D), jnp.bfloat16)],
        compiler_params=pltpu.CompilerParams(
            dimension_semantics=("arbitrary",)),
    )(h_0, w_ab, w_cd)

    return pl.pallas_call(
        _mlp_kernel,
        in_specs=[
            pl.BlockSpec((8, D), lambda: (0, 0)),
            pl.BlockSpec((D, D), lambda: (0, 0)),
            pl.BlockSpec((1, D), lambda: (0, 0)),
            pl.BlockSpec((D, D), lambda: (0, 0)),
            pl.BlockSpec((1, D), lambda: (0, 0)),
            pl.BlockSpec((D, OUT), lambda: (0, 0)),
            pl.BlockSpec((1, OUT), lambda: (0, 0)),
        ],
        out_specs=pl.BlockSpec((B, OUT), lambda: (0, 0)),
        out_shape=jax.ShapeDtypeStruct((B, OUT), jnp.float32),
    )(pooled, C1_w, C1_b.reshape(1, D), C2_w, C2_b.reshape(1, D),
      C3_w, C3_b.reshape(1, OUT))
